# baseline (speedup 1.0000x reference)
import functools

import jax
import jax.numpy as jnp
from jax import lax
from jax.experimental import pallas as pl
from jax.experimental.pallas import tpu as pltpu
from jax.experimental.pallas import tpu_sc as plsc

_N = 50000
_T = 12
_F = 5
_H = 64
_TF = _T * _F
_NP = 51200
_PN = 3200
_E = 800000
_EP = 802816
_ER = _EP // 128
_TR = _ER // 16
_CR = _ER // 32
_NB = 50176
_BN = _NB // 16

_SC_PARAMS = pltpu.CompilerParams(needs_layout_passes=False,
                                  use_tc_tiling_on_sc=False)


def _sca_body(dst_h, ew_h, deg_h,
              degtab, cb, ob, dst2, ew2, zb16, ewrows, ewrows2, dsem0, dsem1):
    c = lax.axis_index("c")
    s = lax.axis_index("s")
    nbase = s * _PN

    def z1(i, carry):
        zb16[i, pl.ds(0, 16)] = jnp.zeros((16,), jnp.float32)
        return carry
    lax.fori_loop(0, 200, z1, 0)
    for r in range(16):
        pltpu.sync_copy(zb16, degtab.at[pl.ds(nbase + r * 200, 200)])
    plsc.subcore_barrier()

    rbase = (c * 16 + s) * _CR

    def degb(sup, carry):
        br = rbase + sup * 4
        pltpu.sync_copy(dst_h.at[pl.ds(br, 4)], dst2)
        pltpu.sync_copy(ew_h.at[pl.ds(br, 4)], ew2)
        sd = [None, None]
        for j in range(4):
            jv = jnp.full((16,), j, jnp.int32)
            eb = ewrows if j % 2 == 0 else ewrows2
            if sd[j % 2] is not None:
                sd[j % 2].wait()

            def bld(r8, carry2, _jv=jv, _eb=eb):
                for u in range(8):
                    r = r8 * 8 + u
                    sp = plsc.load_gather(ew2, [_jv, jnp.broadcast_to(r, (16,))])
                    _eb[r, pl.ds(0, 16)] = sp
                return carry2
            lax.fori_loop(0, 16, bld, 0)
            sd[j % 2] = pltpu.async_copy(eb, degtab.at[dst2.at[j]],
                                         dsem0 if j % 2 == 0 else dsem1,
                                         add=True)
        sd[0].wait()
        sd[1].wait()
        return carry
    lax.fori_loop(0, _CR // 4, degb, 0)
    plsc.subcore_barrier()

    lane = lax.iota(jnp.int32, 16)

    def rchunk(cc, carry):
        pltpu.sync_copy(degtab.at[pl.ds(nbase + cc * 320, 320)], cb)

        def rr(g, carry2):
            idx_rows = jnp.full((16,), g * 16, jnp.int32) + lane
            acc = plsc.load_gather(cb, [idx_rows, jnp.zeros((16,), jnp.int32)])
            ob[pl.ds(g * 16, 16)] = acc
            return carry2
        lax.fori_loop(0, 20, rr, 0)
        pltpu.sync_copy(ob, deg_h.at[c, pl.ds(nbase + cc * 320, 320)])
        return carry
    lax.fori_loop(0, 10, rchunk, 0)


_sc_deg = functools.partial(
    pl.kernel,
    out_type=jax.ShapeDtypeStruct((2, _NP), jnp.float32),
    mesh=plsc.VectorSubcoreMesh(core_axis_name="c", subcore_axis_name="s"),
    compiler_params=_SC_PARAMS,
    scratch_types=[
        pltpu.VMEM_SHARED((_NP, 16), jnp.float32),
        pltpu.VMEM((320, 16), jnp.float32),
        pltpu.VMEM((320,), jnp.float32),
        pltpu.VMEM((4, 128), jnp.int32),
        pltpu.VMEM((4, 128), jnp.float32),
        pltpu.VMEM((200, 16), jnp.float32),
        pltpu.VMEM((128, 16), jnp.float32),
        pltpu.VMEM((128, 16), jnp.float32),
        pltpu.SemaphoreType.DMA,
        pltpu.SemaphoreType.DMA,
    ],
)(_sca_body)


def _tcb_body(dega, degb, x2, x2p_ref, dinv_ref):
    d = dega[0] + degb[0] + 1.0
    dv = jax.lax.rsqrt(d)
    dinv_ref[...] = dv
    x2p_ref[0] = dv * x2[0]


def _tc_prescale(deg3, x2):
    return pl.pallas_call(
        _tcb_body,
        grid=(2, _NB // 512),
        in_specs=[
            pl.BlockSpec((1, 512, 1), lambda c, i: (0, i, 0)),
            pl.BlockSpec((1, 512, 1), lambda c, i: (1, i, 0)),
            pl.BlockSpec((1, 512, 32), lambda c, i: (c, i, 0)),
        ],
        out_specs=[
            pl.BlockSpec((1, 512, 32), lambda c, i: (c, i, 0)),
            pl.BlockSpec((512, 1), lambda c, i: (i, 0)),
        ],
        out_shape=[
            jax.ShapeDtypeStruct((2, _NB, 32), jnp.float32),
            jax.ShapeDtypeStruct((_NB, 1), jnp.float32),
        ],
    )(deg3, deg3, x2)


def _scc_body(src_h, dst_h, ew_h, x2_h, y2_h,
              yacc, src2, dst2, ew2, gidx2, rows0, rows1, rows2, zby,
              gs0, gs1, gs2, ss0, ss1, ss2):
    c = lax.axis_index("c")
    s = lax.axis_index("s")
    abase = s * _BN
    rowsb = (rows0, rows1, rows2)
    gsem = (gs0, gs1, gs2)
    ssem = (ss0, ss1, ss2)

    def z1(i, carry):
        rr = i // 2
        o = (i % 2) * 16
        zby[rr, pl.ds(o, 16)] = jnp.zeros((16,), jnp.float32)
        return carry
    lax.fori_loop(0, 392, z1, 0)
    for r in range(16):
        pltpu.sync_copy(zby, yacc.at[pl.ds(abase + r * 196, 196)])
    plsc.subcore_barrier()

    cN = c * _NB

    def sup_body(sup, carry):
        br = s * _TR + sup * 8
        pltpu.sync_copy(src_h.at[pl.ds(br, 8)], src2)
        pltpu.sync_copy(dst_h.at[pl.ds(br, 8)], dst2)
        pltpu.sync_copy(ew_h.at[pl.ds(br, 8)], ew2)

        def gix(i, carry2):
            j = i // 8
            k = (i % 8) * 16
            gidx2[j, pl.ds(k, 16)] = src2[j, pl.ds(k, 16)] + cN
            return carry2
        lax.fori_loop(0, 64, gix, 0)

        gd = [None] * 8
        sd = [None] * 8
        gd[0] = pltpu.async_copy(x2_h.at[gidx2.at[0]], rows0, gs0)
        gd[1] = pltpu.async_copy(x2_h.at[gidx2.at[1]], rows1, gs1)
        for j in range(8):
            b = j % 3
            gd[j].wait()
            jv = jnp.full((16,), j, jnp.int32)
            rb = rowsb[b]

            def scl(r8, carry2, _jv=jv, _rb=rb):
                for u in range(8):
                    r = r8 * 8 + u
                    sp = plsc.load_gather(ew2, [_jv, jnp.broadcast_to(r, (16,))])
                    _rb[r, pl.ds(0, 16)] = _rb[r, pl.ds(0, 16)] * sp
                    _rb[r, pl.ds(16, 16)] = _rb[r, pl.ds(16, 16)] * sp
                return carry2
            lax.fori_loop(0, 16, scl, 0)
            sd[j] = pltpu.async_copy(rb, yacc.at[dst2.at[j]], ssem[b], add=True)
            if j + 2 < 8:
                if j >= 1:
                    sd[j - 1].wait()
                nb = (j + 2) % 3
                gd[j + 2] = pltpu.async_copy(x2_h.at[gidx2.at[j + 2]],
                                             rowsb[nb], gsem[nb])
        sd[5].wait()
        sd[6].wait()
        sd[7].wait()
        return carry
    lax.fori_loop(0, _NSUP, sup_body, 0)
    plsc.subcore_barrier()

    for r in range(16):
        pltpu.sync_copy(yacc.at[pl.ds(abase + r * 196, 196)],
                        y2_h.at[c, pl.ds(abase + r * 196, 196)])


_NSUP = _TR // 8

_sc_spmm = functools.partial(
    pl.kernel,
    out_type=jax.ShapeDtypeStruct((2, _NB, 32), jnp.float32),
    mesh=plsc.VectorSubcoreMesh(core_axis_name="c", subcore_axis_name="s"),
    compiler_params=_SC_PARAMS,
    scratch_types=[
        pltpu.VMEM_SHARED((_NB, 32), jnp.float32),
        pltpu.VMEM((8, 128), jnp.int32),
        pltpu.VMEM((8, 128), jnp.int32),
        pltpu.VMEM((8, 128), jnp.float32),
        pltpu.VMEM((8, 128), jnp.int32),
        pltpu.VMEM((128, 32), jnp.float32),
        pltpu.VMEM((128, 32), jnp.float32),
        pltpu.VMEM((128, 32), jnp.float32),
        pltpu.VMEM((196, 32), jnp.float32),
        pltpu.SemaphoreType.DMA,
        pltpu.SemaphoreType.DMA,
        pltpu.SemaphoreType.DMA,
        pltpu.SemaphoreType.DMA,
        pltpu.SemaphoreType.DMA,
        pltpu.SemaphoreType.DMA,
    ],
)(_scc_body)


def _tcd_body(ylo, yhi, xb, dv, Bz, Bh, bzr, bhr, misc, f1w, f1b, f2wt,
              out_ref, hacc_ref):
    z = jnp.concatenate([ylo[...], yhi[...]], axis=1)
    d = dv[...]
    y = d * z + (d * d) * xb[...]
    zl = jnp.dot(y, Bz[...], preferred_element_type=jnp.float32) + bzr[...]
    hl = jnp.dot(y, Bh[...], preferred_element_type=jnp.float32) + bhr[...]
    g = (1.0 - jax.nn.sigmoid(zl)) * jnp.tanh(hl)
    acc = misc[0, 0] * g[:, 0:_H]
    for t in range(1, _T):
        acc = acc + misc[0, t] * g[:, t * _H:(t + 1) * _H]
    hacc_ref[...] = acc
    hid = jnp.maximum(jnp.dot(acc, f1w[...], preferred_element_type=jnp.float32)
                      + f1b[...], 0.0)
    out_ref[...] = jnp.sum(hid * f2wt[...], axis=1, keepdims=True) + misc[0, 64]


def _tc_post(ylo, yhi, xb, dv, Bz, Bh, bzr, bhr, misc, f1w, f1b, f2wt):
    return pl.pallas_call(
        _tcd_body,
        grid=(_NB // 512,),
        in_specs=[
            pl.BlockSpec((512, 32), lambda i: (i, 0)),
            pl.BlockSpec((512, 32), lambda i: (i, 0)),
            pl.BlockSpec((512, 64), lambda i: (i, 0)),
            pl.BlockSpec((512, 1), lambda i: (i, 0)),
            pl.BlockSpec((64, _T * _H), lambda i: (0, 0)),
            pl.BlockSpec((64, _T * _H), lambda i: (0, 0)),
            pl.BlockSpec((1, _T * _H), lambda i: (0, 0)),
            pl.BlockSpec((1, _T * _H), lambda i: (0, 0)),
            pl.BlockSpec((1, 128), lambda i: (0, 0)),
            pl.BlockSpec((64, 32), lambda i: (0, 0)),
            pl.BlockSpec((1, 32), lambda i: (0, 0)),
            pl.BlockSpec((1, 32), lambda i: (0, 0)),
        ],
        out_specs=[
            pl.BlockSpec((512, 1), lambda i: (i, 0)),
            pl.BlockSpec((512, _H), lambda i: (i, 0)),
        ],
        out_shape=[
            jax.ShapeDtypeStruct((_NB, 1), jnp.float32),
            jax.ShapeDtypeStruct((_NB, _H), jnp.float32),
        ],
    )(ylo, yhi, xb, dv, Bz, Bh, bzr, bhr, misc, f1w, f1b, f2wt)


def kernel(x, edge_index, edge_weight, attention, Wz, bz, Wr, br, Wh, bh,
           Wlz, blz, Wlr, blr, Wlh, blh, fc1_w, fc1_b, fc2_w, fc2_b):
    src = edge_index[0].astype(jnp.int32)
    dst = edge_index[1].astype(jnp.int32)
    ew = edge_weight.astype(jnp.float32)

    epad = _EP - _E
    src2d = jnp.pad(src, (0, epad)).reshape(_ER, 128)
    dst2d = jnp.pad(dst, (0, epad)).reshape(_ER, 128)
    ew2d = jnp.pad(ew, (0, epad)).reshape(_ER, 128)

    xf = x.reshape(_N, _TF)
    xbp = jnp.pad(xf, ((0, _NB - _N), (0, 64 - _TF)))
    x2 = jnp.stack([xbp[:, :32], xbp[:, 32:]])

    deg2 = _sc_deg(dst2d, ew2d)
    deg3 = deg2[:, :_NB, None]
    x2p, dinv = _tc_prescale(deg3, x2)
    y2 = _sc_spmm(src2d, dst2d, ew2d, x2p.reshape(2 * _NB, 32))

    Mz = Wz @ Wlz[:_H]
    Mh = Wh @ Wlh[:_H]
    bzc = bz @ Wlz[:_H] + blz
    bhc = bh @ Wlh[:_H] + blh
    eye_t = jnp.eye(_T, dtype=jnp.float32)
    Bz = jnp.pad(jnp.kron(eye_t, Mz), ((0, 4), (0, 0)))
    Bh = jnp.pad(jnp.kron(eye_t, Mh), ((0, 4), (0, 0)))
    bzr = jnp.tile(bzc, _T)[None, :]
    bhr = jnp.tile(bhc, _T)[None, :]
    probs = jax.nn.softmax(attention)
    misc = jnp.zeros((1, 128), jnp.float32)
    misc = misc.at[0, :_T].set(probs).at[0, 64].set(fc2_b[0])
    f1b = fc1_b[None, :]
    f2wt = fc2_w[:, 0][None, :]

    out_p, hacc_p = _tc_post(y2[0], y2[1], xbp, dinv,
                             Bz, Bh, bzr, bhr, misc, fc1_w, f1b, f2wt)
    return out_p[:_N], hacc_p[:_N]


# speedup vs baseline: 1.0008x; 1.0008x over previous
import functools

import jax
import jax.numpy as jnp
from jax import lax
from jax.experimental import pallas as pl
from jax.experimental.pallas import tpu as pltpu
from jax.experimental.pallas import tpu_sc as plsc

_N = 50000
_T = 12
_F = 5
_H = 64
_TF = _T * _F
_NP = 51200
_PN = 3200
_E = 800000
_EP = 802816
_ER = _EP // 128
_TR = _ER // 16
_CR = _ER // 32
_NB = 50176
_BN = _NB // 16

_SC_PARAMS = pltpu.CompilerParams(needs_layout_passes=False,
                                  use_tc_tiling_on_sc=False)


def _sca_body(dst_h, ew_h, deg_h,
              degtab, cb, ob, dst2, ew2, zb16, ewrows, ewrows2, dsem0, dsem1):
    c = lax.axis_index("c")
    s = lax.axis_index("s")
    nbase = s * _PN

    def z1(i, carry):
        zb16[i, pl.ds(0, 16)] = jnp.zeros((16,), jnp.float32)
        return carry
    lax.fori_loop(0, 200, z1, 0)
    for r in range(16):
        pltpu.sync_copy(zb16, degtab.at[pl.ds(nbase + r * 200, 200)])
    plsc.subcore_barrier()

    rbase = (c * 16 + s) * _CR

    def degb(sup, carry):
        br = rbase + sup * 4
        pltpu.sync_copy(dst_h.at[pl.ds(br, 4)], dst2)
        pltpu.sync_copy(ew_h.at[pl.ds(br, 4)], ew2)
        sd = [None, None]
        for j in range(4):
            jv = jnp.full((16,), j, jnp.int32)
            eb = ewrows if j % 2 == 0 else ewrows2
            if sd[j % 2] is not None:
                sd[j % 2].wait()

            def bld(r8, carry2, _jv=jv, _eb=eb):
                for u in range(8):
                    r = r8 * 8 + u
                    sp = plsc.load_gather(ew2, [_jv, jnp.broadcast_to(r, (16,))])
                    _eb[r, pl.ds(0, 16)] = sp
                return carry2
            lax.fori_loop(0, 16, bld, 0)
            sd[j % 2] = pltpu.async_copy(eb, degtab.at[dst2.at[j]],
                                         dsem0 if j % 2 == 0 else dsem1,
                                         add=True)
        sd[0].wait()
        sd[1].wait()
        return carry
    lax.fori_loop(0, _CR // 4, degb, 0)
    plsc.subcore_barrier()

    lane = lax.iota(jnp.int32, 16)

    def rchunk(cc, carry):
        pltpu.sync_copy(degtab.at[pl.ds(nbase + cc * 320, 320)], cb)

        def rr(g, carry2):
            idx_rows = jnp.full((16,), g * 16, jnp.int32) + lane
            acc = plsc.load_gather(cb, [idx_rows, jnp.zeros((16,), jnp.int32)])
            ob[pl.ds(g * 16, 16)] = acc
            return carry2
        lax.fori_loop(0, 20, rr, 0)
        pltpu.sync_copy(ob, deg_h.at[c, pl.ds(nbase + cc * 320, 320)])
        return carry
    lax.fori_loop(0, 10, rchunk, 0)


_sc_deg = functools.partial(
    pl.kernel,
    out_type=jax.ShapeDtypeStruct((2, _NP), jnp.float32),
    mesh=plsc.VectorSubcoreMesh(core_axis_name="c", subcore_axis_name="s"),
    compiler_params=_SC_PARAMS,
    scratch_types=[
        pltpu.VMEM_SHARED((_NP, 16), jnp.float32),
        pltpu.VMEM((320, 16), jnp.float32),
        pltpu.VMEM((320,), jnp.float32),
        pltpu.VMEM((4, 128), jnp.int32),
        pltpu.VMEM((4, 128), jnp.float32),
        pltpu.VMEM((200, 16), jnp.float32),
        pltpu.VMEM((128, 16), jnp.float32),
        pltpu.VMEM((128, 16), jnp.float32),
        pltpu.SemaphoreType.DMA,
        pltpu.SemaphoreType.DMA,
    ],
)(_sca_body)


def _tcb_body(dega, degb, x2, x2p_ref, dinv_ref):
    d = dega[0] + degb[0] + 1.0
    dv = jax.lax.rsqrt(d)
    dinv_ref[...] = dv
    x2p_ref[0] = dv * x2[0]


def _tc_prescale(deg3, x2):
    return pl.pallas_call(
        _tcb_body,
        grid=(2, _NB // 512),
        in_specs=[
            pl.BlockSpec((1, 512, 1), lambda c, i: (0, i, 0)),
            pl.BlockSpec((1, 512, 1), lambda c, i: (1, i, 0)),
            pl.BlockSpec((1, 512, 32), lambda c, i: (c, i, 0)),
        ],
        out_specs=[
            pl.BlockSpec((1, 512, 32), lambda c, i: (c, i, 0)),
            pl.BlockSpec((512, 1), lambda c, i: (i, 0)),
        ],
        out_shape=[
            jax.ShapeDtypeStruct((2, _NB, 32), jnp.float32),
            jax.ShapeDtypeStruct((_NB, 1), jnp.float32),
        ],
    )(deg3, deg3, x2)


def _scc_body(src_h, dst_h, ew_h, x2_h, y2_h,
              yacc, src2, dst2, ew2, gidx2, rows0, rows1, rows2, rows3, zby,
              gs0, gs1, gs2, gs3, ss0, ss1, ss2, ss3):
    c = lax.axis_index("c")
    s = lax.axis_index("s")
    abase = s * _BN
    rowsb = (rows0, rows1, rows2, rows3)
    gsem = (gs0, gs1, gs2, gs3)
    ssem = (ss0, ss1, ss2, ss3)

    def z1(i, carry):
        rr = i // 2
        o = (i % 2) * 16
        zby[rr, pl.ds(o, 16)] = jnp.zeros((16,), jnp.float32)
        return carry
    lax.fori_loop(0, 392, z1, 0)
    for r in range(16):
        pltpu.sync_copy(zby, yacc.at[pl.ds(abase + r * 196, 196)])
    plsc.subcore_barrier()

    cN = c * _NB

    def sup_body(sup, carry):
        br = s * _TR + sup * 8
        pltpu.sync_copy(src_h.at[pl.ds(br, 8)], src2)
        pltpu.sync_copy(dst_h.at[pl.ds(br, 8)], dst2)
        pltpu.sync_copy(ew_h.at[pl.ds(br, 8)], ew2)

        def gix(i, carry2):
            j = i // 8
            k = (i % 8) * 16
            gidx2[j, pl.ds(k, 16)] = src2[j, pl.ds(k, 16)] + cN
            return carry2
        lax.fori_loop(0, 64, gix, 0)

        gd = [None] * 8
        sd = [None] * 8
        gd[0] = pltpu.async_copy(x2_h.at[gidx2.at[0]], rows0, gs0)
        gd[1] = pltpu.async_copy(x2_h.at[gidx2.at[1]], rows1, gs1)
        gd[2] = pltpu.async_copy(x2_h.at[gidx2.at[2]], rows2, gs2)
        for j in range(8):
            b = j % 4
            gd[j].wait()
            jv = jnp.full((16,), j, jnp.int32)
            rb = rowsb[b]

            def scl(r8, carry2, _jv=jv, _rb=rb):
                for u in range(8):
                    r = r8 * 8 + u
                    sp = plsc.load_gather(ew2, [_jv, jnp.broadcast_to(r, (16,))])
                    _rb[r, pl.ds(0, 16)] = _rb[r, pl.ds(0, 16)] * sp
                    _rb[r, pl.ds(16, 16)] = _rb[r, pl.ds(16, 16)] * sp
                return carry2
            lax.fori_loop(0, 16, scl, 0)
            sd[j] = pltpu.async_copy(rb, yacc.at[dst2.at[j]], ssem[b], add=True)
            if j + 3 < 8:
                if j >= 1:
                    sd[j - 1].wait()
                nb = (j + 3) % 4
                gd[j + 3] = pltpu.async_copy(x2_h.at[gidx2.at[j + 3]],
                                             rowsb[nb], gsem[nb])
        sd[4].wait()
        sd[5].wait()
        sd[6].wait()
        sd[7].wait()
        return carry
    lax.fori_loop(0, _NSUP, sup_body, 0)
    plsc.subcore_barrier()

    for r in range(16):
        pltpu.sync_copy(yacc.at[pl.ds(abase + r * 196, 196)],
                        y2_h.at[c, pl.ds(abase + r * 196, 196)])


_NSUP = _TR // 8

_sc_spmm = functools.partial(
    pl.kernel,
    out_type=jax.ShapeDtypeStruct((2, _NB, 32), jnp.float32),
    mesh=plsc.VectorSubcoreMesh(core_axis_name="c", subcore_axis_name="s"),
    compiler_params=_SC_PARAMS,
    scratch_types=[
        pltpu.VMEM_SHARED((_NB, 32), jnp.float32),
        pltpu.VMEM((8, 128), jnp.int32),
        pltpu.VMEM((8, 128), jnp.int32),
        pltpu.VMEM((8, 128), jnp.float32),
        pltpu.VMEM((8, 128), jnp.int32),
        pltpu.VMEM((128, 32), jnp.float32),
        pltpu.VMEM((128, 32), jnp.float32),
        pltpu.VMEM((128, 32), jnp.float32),
        pltpu.VMEM((128, 32), jnp.float32),
        pltpu.VMEM((196, 32), jnp.float32),
        pltpu.SemaphoreType.DMA,
        pltpu.SemaphoreType.DMA,
        pltpu.SemaphoreType.DMA,
        pltpu.SemaphoreType.DMA,
        pltpu.SemaphoreType.DMA,
        pltpu.SemaphoreType.DMA,
        pltpu.SemaphoreType.DMA,
        pltpu.SemaphoreType.DMA,
    ],
)(_scc_body)


def _tcd_body(ylo, yhi, xb, dv, Bz, Bh, bzr, bhr, misc, f1w, f1b, f2wt,
              out_ref, hacc_ref):
    z = jnp.concatenate([ylo[...], yhi[...]], axis=1)
    d = dv[...]
    y = d * z + (d * d) * xb[...]
    zl = jnp.dot(y, Bz[...], preferred_element_type=jnp.float32) + bzr[...]
    hl = jnp.dot(y, Bh[...], preferred_element_type=jnp.float32) + bhr[...]
    g = (1.0 - jax.nn.sigmoid(zl)) * jnp.tanh(hl)
    acc = misc[0, 0] * g[:, 0:_H]
    for t in range(1, _T):
        acc = acc + misc[0, t] * g[:, t * _H:(t + 1) * _H]
    hacc_ref[...] = acc
    hid = jnp.maximum(jnp.dot(acc, f1w[...], preferred_element_type=jnp.float32)
                      + f1b[...], 0.0)
    out_ref[...] = jnp.sum(hid * f2wt[...], axis=1, keepdims=True) + misc[0, 64]


def _tc_post(ylo, yhi, xb, dv, Bz, Bh, bzr, bhr, misc, f1w, f1b, f2wt):
    return pl.pallas_call(
        _tcd_body,
        grid=(_NB // 512,),
        in_specs=[
            pl.BlockSpec((512, 32), lambda i: (i, 0)),
            pl.BlockSpec((512, 32), lambda i: (i, 0)),
            pl.BlockSpec((512, 64), lambda i: (i, 0)),
            pl.BlockSpec((512, 1), lambda i: (i, 0)),
            pl.BlockSpec((64, _T * _H), lambda i: (0, 0)),
            pl.BlockSpec((64, _T * _H), lambda i: (0, 0)),
            pl.BlockSpec((1, _T * _H), lambda i: (0, 0)),
            pl.BlockSpec((1, _T * _H), lambda i: (0, 0)),
            pl.BlockSpec((1, 128), lambda i: (0, 0)),
            pl.BlockSpec((64, 32), lambda i: (0, 0)),
            pl.BlockSpec((1, 32), lambda i: (0, 0)),
            pl.BlockSpec((1, 32), lambda i: (0, 0)),
        ],
        out_specs=[
            pl.BlockSpec((512, 1), lambda i: (i, 0)),
            pl.BlockSpec((512, _H), lambda i: (i, 0)),
        ],
        out_shape=[
            jax.ShapeDtypeStruct((_NB, 1), jnp.float32),
            jax.ShapeDtypeStruct((_NB, _H), jnp.float32),
        ],
    )(ylo, yhi, xb, dv, Bz, Bh, bzr, bhr, misc, f1w, f1b, f2wt)


def kernel(x, edge_index, edge_weight, attention, Wz, bz, Wr, br, Wh, bh,
           Wlz, blz, Wlr, blr, Wlh, blh, fc1_w, fc1_b, fc2_w, fc2_b):
    src = edge_index[0].astype(jnp.int32)
    dst = edge_index[1].astype(jnp.int32)
    ew = edge_weight.astype(jnp.float32)

    epad = _EP - _E
    src2d = jnp.pad(src, (0, epad)).reshape(_ER, 128)
    dst2d = jnp.pad(dst, (0, epad)).reshape(_ER, 128)
    ew2d = jnp.pad(ew, (0, epad)).reshape(_ER, 128)

    xf = x.reshape(_N, _TF)
    xbp = jnp.pad(xf, ((0, _NB - _N), (0, 64 - _TF)))
    x2 = jnp.stack([xbp[:, :32], xbp[:, 32:]])

    deg2 = _sc_deg(dst2d, ew2d)
    deg3 = deg2[:, :_NB, None]
    x2p, dinv = _tc_prescale(deg3, x2)
    y2 = _sc_spmm(src2d, dst2d, ew2d, x2p.reshape(2 * _NB, 32))

    Mz = Wz @ Wlz[:_H]
    Mh = Wh @ Wlh[:_H]
    bzc = bz @ Wlz[:_H] + blz
    bhc = bh @ Wlh[:_H] + blh
    eye_t = jnp.eye(_T, dtype=jnp.float32)
    Bz = jnp.pad(jnp.kron(eye_t, Mz), ((0, 4), (0, 0)))
    Bh = jnp.pad(jnp.kron(eye_t, Mh), ((0, 4), (0, 0)))
    bzr = jnp.tile(bzc, _T)[None, :]
    bhr = jnp.tile(bhc, _T)[None, :]
    probs = jax.nn.softmax(attention)
    misc = jnp.zeros((1, 128), jnp.float32)
    misc = misc.at[0, :_T].set(probs).at[0, 64].set(fc2_b[0])
    f1b = fc1_b[None, :]
    f2wt = fc2_w[:, 0][None, :]

    out_p, hacc_p = _tc_post(y2[0], y2[1], xbp, dinv,
                             Bz, Bh, bzr, bhr, misc, fc1_w, f1b, f2wt)
    return out_p[:_N], hacc_p[:_N]
